# baseline (speedup 1.0000x reference)
import functools

import jax
import jax.numpy as jnp
from jax import lax
from jax.experimental import pallas as pl
from jax.experimental.pallas import tpu as pltpu
from jax.experimental.pallas import tpu_sc as plsc

_VOCAB = 100000
_EMBED = 64
_BATCH = 1024

_BV = 1024
_NB = 6
_STEPS = pl.cdiv(_VOCAB, _BV)
_TAIL = _VOCAB - (_STEPS - 1) * _BV

def _make_sc_gather(B, D):
    info = plsc.get_sparse_core_info()
    nw = info.num_cores * info.num_subcores
    b_per_w = B // nw
    mesh = plsc.VectorSubcoreMesh(core_axis_name="c", subcore_axis_name="s")

    @functools.partial(
        pl.kernel,
        mesh=mesh,
        out_type=jax.ShapeDtypeStruct((B, D), jnp.float32),
        scratch_types=[
            pltpu.VMEM((b_per_w,), jnp.int32),
            pltpu.VMEM((b_per_w,), jnp.int32),
            pltpu.VMEM((b_per_w, D), jnp.float32),
            pltpu.SemaphoreType.DMA,
        ],
        compiler_params=pltpu.CompilerParams(use_tc_tiling_on_sc=True),
    )
    def gather_rows(idx_hbm, table_hbm, out_hbm, idx_v, pair_v, rows_v, sem):
        wid = lax.axis_index("s") * info.num_cores + lax.axis_index("c")
        base = wid * b_per_w
        pltpu.sync_copy(idx_hbm.at[pl.ds(base, b_per_w)], idx_v)
        for i in range(b_per_w // 16):
            sl = pl.ds(i * 16, 16)
            v = idx_v[sl]
            pair_v[sl] = (
                lax.shift_left(lax.shift_right_logical(v, 11), 10)
                | (v & 1023)
            )
        pltpu.async_copy(table_hbm.at[pair_v], rows_v, sem).wait()
        pltpu.sync_copy(rows_v, out_hbm.at[pl.ds(base, b_per_w)])

    return gather_rows


_sc_gather = _make_sc_gather(_BATCH, 128)

_PAIRS = 50176


def _mm_body(ctr_ref, emb_ref, wt_ref, out_hbm, embt_s, scr, sems):
    j = pl.program_id(0)
    jm = lax.rem(j, _NB)

    @pl.when(j == 0)
    def _tr():
        par = (lax.shift_right_logical(ctr_ref[...], 10) & 1).astype(
            jnp.float32
        )
        emb64 = (emb_ref[:, pl.ds(0, _EMBED)] * (1.0 - par)
                 + emb_ref[:, pl.ds(_EMBED, _EMBED)] * par)
        embt_s[...] = emb64.T

    @pl.when(j >= _NB)
    def _wait_reuse():
        pltpu.make_async_copy(
            scr.at[jm], out_hbm.at[pl.ds(0, _BV), :], sems.at[jm]
        ).wait()

    scr[jm] = lax.dot_general(
        wt_ref[...], embt_s[...],
        dimension_numbers=(((0,), (0,)), ((), ())),
        preferred_element_type=jnp.float32,
    )

    @pl.when(j < _STEPS - 1)
    def _start_full():
        pltpu.make_async_copy(
            scr.at[jm], out_hbm.at[pl.ds(j * _BV, _BV), :], sems.at[jm]
        ).start()

    @pl.when(j == _STEPS - 1)
    def _tail_and_drain():
        tm = (_STEPS - 1) % _NB
        base = (_STEPS - 1) * _BV
        pltpu.make_async_copy(
            scr.at[tm, pl.ds(0, _TAIL), :],
            out_hbm.at[pl.ds(base, _TAIL), :],
            sems.at[tm],
        ).start()
        for k in range(_NB):
            if k == tm:
                pltpu.make_async_copy(
                    scr.at[k, pl.ds(0, _TAIL), :],
                    out_hbm.at[pl.ds(0, _TAIL), :],
                    sems.at[k],
                ).wait()
            else:
                pltpu.make_async_copy(
                    scr.at[k], out_hbm.at[pl.ds(0, _BV), :], sems.at[k]
                ).wait()


def _tc_project(ctr, emb, wt):
    return pl.pallas_call(
        _mm_body,
        grid=(_STEPS,),
        in_specs=[
            pl.BlockSpec((_BATCH, 1), lambda j: (0, 0)),
            pl.BlockSpec((_BATCH, 128), lambda j: (0, 0)),
            pl.BlockSpec((_EMBED, _BV), lambda j: (0, j)),
        ],
        out_specs=pl.BlockSpec(memory_space=pl.ANY),
        out_shape=jax.ShapeDtypeStruct((_VOCAB, _BATCH), jnp.float32),
        scratch_shapes=[
            pltpu.VMEM((_EMBED, _BATCH), jnp.float32),
            pltpu.VMEM((_NB, _BV, _BATCH), jnp.float32),
            pltpu.SemaphoreType.DMA((_NB,)),
        ],
        compiler_params=pltpu.CompilerParams(
            dimension_semantics=("arbitrary",),
            fuse_transposed_lhs_in_matmul=True,
        ),
    )(ctr, emb, wt)


_BT = 2048


_NPB = 4
_PSTEPS = _PAIRS // 1024


def _tp_body(tt_ref, out_hbm, scr, sems):
    j = pl.program_id(0)
    jm = lax.rem(j, _NPB)

    @pl.when(j >= _NPB)
    def _wait_reuse():
        pltpu.make_async_copy(
            scr.at[jm], out_hbm.at[pl.ds(0, 1024), :], sems.at[jm]
        ).wait()

    scr[jm, :, pl.ds(0, _EMBED)] = tt_ref[:, pl.ds(0, 1024)].T
    scr[jm, :, pl.ds(_EMBED, _EMBED)] = tt_ref[:, pl.ds(1024, 1024)].T
    pltpu.make_async_copy(
        scr.at[jm], out_hbm.at[pl.ds(j * 1024, 1024), :], sems.at[jm]
    ).start()

    @pl.when(j == _PSTEPS - 1)
    def _drain():
        for k in range(_NPB):
            pltpu.make_async_copy(
                scr.at[k], out_hbm.at[pl.ds(0, 1024), :], sems.at[k]
            ).wait()


def _tc_transpose_pack(tt):
    return pl.pallas_call(
        _tp_body,
        grid=(_PSTEPS,),
        in_specs=[
            pl.BlockSpec((_EMBED, 2048), lambda j: (0, j)),
        ],
        out_specs=pl.BlockSpec(memory_space=pl.ANY),
        out_shape=jax.ShapeDtypeStruct((_PAIRS, 128), jnp.float32),
        scratch_shapes=[
            pltpu.VMEM((_NPB, 1024, 128), jnp.float32),
            pltpu.SemaphoreType.DMA((_NPB,)),
        ],
        compiler_params=pltpu.CompilerParams(
            dimension_semantics=("arbitrary",),
        ),
    )(tt)


def kernel(center, emb_table, w):
    pairs = _tc_transpose_pack(emb_table.T)
    emb = _sc_gather(center, pairs)
    outT = _tc_project(center.reshape(_BATCH, 1), emb, w.T)
    return outT.T
